# Initial kernel scaffold; baseline (speedup 1.0000x reference)
import jax
import jax.numpy as jnp
from jax.experimental import pallas as pl


def kernel(x, edge_index, W1, b1, W2, b2):
    raise NotImplementedError("write your pallas kernel here")


# speedup vs baseline: 34.7674x; 34.7674x over previous
import functools

import jax
import jax.numpy as jnp
from jax import lax
from jax.experimental import pallas as pl
from jax.experimental.pallas import tpu as pltpu
from jax.experimental.pallas import tpu_sc as plsc

NC = 2
NS = 16
NW = NC * NS
CHUNK = 128

_F32 = jnp.float32


def _sc_mesh():
    return plsc.VectorSubcoreMesh(
        core_axis_name="c", subcore_axis_name="s", num_cores=NC, num_subcores=NS
    )


@functools.lru_cache(maxsize=None)
def _make_deg_kernel(ch: int, npad: int):
    rows = npad // NS

    def body(dst_hbm, zeros_hbm, ones_hbm, out_hbm, dstv, onesv, deg_sh):
        c = lax.axis_index("c")
        s = lax.axis_index("s")
        w = c * NS + s
        pltpu.sync_copy(zeros_hbm.at[pl.ds(s * rows, rows)],
                        deg_sh.at[pl.ds(s * rows, rows)])
        pltpu.sync_copy(ones_hbm, onesv)
        pltpu.sync_copy(dst_hbm.at[w], dstv)
        plsc.subcore_barrier()

        @pl.loop(0, ch)
        def _(j):
            pltpu.sync_copy(onesv, deg_sh.at[dstv.at[j]], add=True)

        plsc.subcore_barrier()
        pltpu.sync_copy(deg_sh.at[pl.ds(s * rows, rows)], out_hbm.at[c, s])

    return pl.kernel(
        body,
        out_type=jax.ShapeDtypeStruct((NC, NS, rows), _F32),
        mesh=_sc_mesh(),
        scratch_types=[
            pltpu.VMEM((ch, CHUNK), jnp.int32),
            pltpu.VMEM((CHUNK,), _F32),
            pltpu.VMEM_SHARED((npad,), _F32),
        ],
    )


@functools.lru_cache(maxsize=None)
def _make_agg_kernel(ch: int, npad: int, feat: int):
    rows = npad // NS
    vec = feat > 1

    def body(src_hbm, dst_hbm, y_hbm, zeros_hbm, out_hbm,
             srcv, dstv, rowsv, agg_sh, sem):
        c = lax.axis_index("c")
        s = lax.axis_index("s")
        w = c * NS + s
        pltpu.sync_copy(zeros_hbm.at[pl.ds(s * rows, rows)],
                        agg_sh.at[pl.ds(s * rows, rows)])
        pltpu.sync_copy(src_hbm.at[w], srcv)
        pltpu.sync_copy(dst_hbm.at[w], dstv)
        plsc.subcore_barrier()

        @pl.loop(0, ch)
        def _(j):
            pltpu.async_copy(y_hbm.at[srcv.at[j]], rowsv, sem).wait()
            pltpu.sync_copy(rowsv, agg_sh.at[dstv.at[j]], add=True)

        plsc.subcore_barrier()
        pltpu.sync_copy(agg_sh.at[pl.ds(s * rows, rows)], out_hbm.at[c, s])

    out_shape = (NC, NS, rows, feat) if vec else (NC, NS, rows)
    return pl.kernel(
        body,
        out_type=jax.ShapeDtypeStruct(out_shape, _F32),
        mesh=_sc_mesh(),
        scratch_types=[
            pltpu.VMEM((ch, CHUNK), jnp.int32),
            pltpu.VMEM((ch, CHUNK), jnp.int32),
            pltpu.VMEM((CHUNK, feat) if vec else (CHUNK,), _F32),
            pltpu.VMEM_SHARED((npad, feat) if vec else (npad,), _F32),
            pltpu.SemaphoreType.DMA,
        ],
        compiler_params=pltpu.CompilerParams(use_tc_tiling_on_sc=False),
    )


def _dense1_body(x_ref, degp_ref, w1_ref, y_ref, dis_ref):
    deg = degp_ref[0] + degp_ref[1] + 1.0
    dis = lax.rsqrt(deg)
    xw = jnp.dot(x_ref[...], w1_ref[...],
                 preferred_element_type=_F32,
                 precision=lax.Precision.HIGHEST)
    dis_ref[...] = dis
    y_ref[...] = dis * xw


def _dense2_body(aggp_ref, y1_ref, dis_ref, b1_ref, w2_ref, y2_ref):
    agg = aggp_ref[0] + aggp_ref[1] + y1_ref[...]
    out1 = dis_ref[...] * agg + b1_ref[...]
    h = jnp.maximum(out1, 0.0)
    hw2 = jnp.sum(h * w2_ref[...], axis=1, keepdims=True)
    y2_ref[...] = dis_ref[...] * hw2


def _final_body(agg2p_ref, y2_ref, dis_ref, b2_ref, out_ref):
    agg = agg2p_ref[0] + agg2p_ref[1] + y2_ref[...]
    out_ref[...] = dis_ref[...] * agg + b2_ref[...]


def _tc_call(body, out_shapes):
    return pl.pallas_call(body, out_shape=out_shapes)


def kernel(x, edge_index, W1, b1, W2, b2):
    n, d = x.shape
    h = W1.shape[1]
    e = edge_index.shape[1]

    npad = ((n + NS * CHUNK - 1) // (NS * CHUNK)) * (NS * CHUNK)
    ch = (e + NW * CHUNK - 1) // (NW * CHUNK)
    ep = NW * ch * CHUNK

    src = edge_index[0]
    dst = edge_index[1]
    if ep > e:
        pad = jnp.full((ep - e,), n, dtype=jnp.int32)
        src = jnp.concatenate([src, pad])
        dst = jnp.concatenate([dst, pad])
    src3 = src.reshape(NW, ch, CHUNK)
    dst3 = dst.reshape(NW, ch, CHUNK)

    xp = jnp.pad(x, ((0, npad - n), (0, 0)))
    zeros1 = jnp.zeros((npad,), _F32)
    zerosf = jnp.zeros((npad, h), _F32)
    ones = jnp.ones((CHUNK,), _F32)

    degp = _make_deg_kernel(ch, npad)(dst3, zeros1, ones)
    degp = degp.reshape(NC, npad, 1)

    y1, dis = _tc_call(_dense1_body, [
        jax.ShapeDtypeStruct((npad, h), _F32),
        jax.ShapeDtypeStruct((npad, 1), _F32),
    ])(xp, degp, W1)

    aggp = _make_agg_kernel(ch, npad, h)(src3, dst3, y1, zerosf)
    aggp = aggp.reshape(NC, npad, h)

    (y2,) = _tc_call(_dense2_body, [jax.ShapeDtypeStruct((npad, 1), _F32)])(
        aggp, y1, dis, b1.reshape(1, h), W2.reshape(1, h))

    agg2p = _make_agg_kernel(ch, npad, 1)(src3, dst3, y2.reshape(npad), zeros1)
    agg2p = agg2p.reshape(NC, npad, 1)

    (out,) = _tc_call(_final_body, [jax.ShapeDtypeStruct((npad, 1), _F32)])(
        agg2p, y2, dis, b2.reshape(1, 1))

    return out[:n, 0]


# speedup vs baseline: 40.7916x; 1.1733x over previous
import functools

import jax
import jax.numpy as jnp
from jax import lax
from jax.experimental import pallas as pl
from jax.experimental.pallas import tpu as pltpu
from jax.experimental.pallas import tpu_sc as plsc

NC = 2
NS = 16
NW = NC * NS
CHUNK = 128

_F32 = jnp.float32


def _sc_mesh():
    return plsc.VectorSubcoreMesh(
        core_axis_name="c", subcore_axis_name="s", num_cores=NC, num_subcores=NS
    )


@functools.lru_cache(maxsize=None)
def _make_deg_kernel(ch: int, npad: int):
    rows = npad // NS

    def body(dst_hbm, zeros_hbm, ones_hbm, out_hbm, dstv, onesv, deg_sh, sem):
        c = lax.axis_index("c")
        s = lax.axis_index("s")
        w = c * NS + s
        pltpu.sync_copy(zeros_hbm.at[pl.ds(s * rows, rows)],
                        deg_sh.at[pl.ds(s * rows, rows)])
        pltpu.sync_copy(ones_hbm, onesv)
        pltpu.sync_copy(dst_hbm.at[w], dstv)
        plsc.subcore_barrier()

        @pl.loop(0, ch // KGRP)
        def _(g):
            descs = [
                pltpu.async_copy(onesv, deg_sh.at[dstv.at[g * KGRP + b]],
                                 sem, add=True)
                for b in range(KGRP)
            ]
            for d in descs:
                d.wait()

        plsc.subcore_barrier()
        pltpu.sync_copy(deg_sh.at[pl.ds(s * rows, rows)], out_hbm.at[c, s])

    return pl.kernel(
        body,
        out_type=jax.ShapeDtypeStruct((NC, NS, rows), _F32),
        mesh=_sc_mesh(),
        scratch_types=[
            pltpu.VMEM((ch, CHUNK), jnp.int32),
            pltpu.VMEM((CHUNK,), _F32),
            pltpu.VMEM_SHARED((npad,), _F32),
            pltpu.SemaphoreType.DMA,
        ],
    )


KGRP = 16


@functools.lru_cache(maxsize=None)
def _make_agg_kernel(ch: int, npad: int, feat: int):
    rows = npad // NS
    vec = feat > 1
    ng = ch // KGRP
    assert ch % KGRP == 0

    def body(src_hbm, dst_hbm, y_hbm, zeros_hbm, out_hbm,
             srcv, dstv, rowsv, agg_sh, gsem, ssem):
        c = lax.axis_index("c")
        s = lax.axis_index("s")
        w = c * NS + s
        pltpu.sync_copy(zeros_hbm.at[pl.ds(s * rows, rows)],
                        agg_sh.at[pl.ds(s * rows, rows)])
        pltpu.sync_copy(src_hbm.at[w], srcv)
        pltpu.sync_copy(dst_hbm.at[w], dstv)
        plsc.subcore_barrier()

        @pl.loop(0, ng)
        def _(g):
            gds = [
                pltpu.async_copy(y_hbm.at[srcv.at[g * KGRP + b]],
                                 rowsv.at[b], gsem)
                for b in range(KGRP)
            ]
            sds = []
            for b in range(KGRP):
                gds[b].wait()
                sds.append(
                    pltpu.async_copy(rowsv.at[b],
                                     agg_sh.at[dstv.at[g * KGRP + b]],
                                     ssem, add=True))
            for d in sds:
                d.wait()

        plsc.subcore_barrier()
        pltpu.sync_copy(agg_sh.at[pl.ds(s * rows, rows)], out_hbm.at[c, s])

    out_shape = (NC, NS, rows, feat) if vec else (NC, NS, rows)
    return pl.kernel(
        body,
        out_type=jax.ShapeDtypeStruct(out_shape, _F32),
        mesh=_sc_mesh(),
        scratch_types=[
            pltpu.VMEM((ch, CHUNK), jnp.int32),
            pltpu.VMEM((ch, CHUNK), jnp.int32),
            pltpu.VMEM((KGRP, CHUNK, feat) if vec else (KGRP, CHUNK), _F32),
            pltpu.VMEM_SHARED((npad, feat) if vec else (npad,), _F32),
            pltpu.SemaphoreType.DMA,
            pltpu.SemaphoreType.DMA,
        ],
        compiler_params=pltpu.CompilerParams(use_tc_tiling_on_sc=False),
    )


def _dense1_body(x_ref, degp_ref, w1_ref, y_ref, dis_ref):
    deg = degp_ref[0] + degp_ref[1] + 1.0
    dis = lax.rsqrt(deg)
    xw = jnp.dot(x_ref[...], w1_ref[...])
    dis_ref[...] = dis
    y_ref[...] = dis * xw


def _dense2_body(aggp_ref, y1_ref, dis_ref, b1_ref, w2_ref, y2_ref):
    agg = aggp_ref[0] + aggp_ref[1] + y1_ref[...]
    out1 = dis_ref[...] * agg + b1_ref[...]
    h = jnp.maximum(out1, 0.0)
    hw2 = jnp.dot(h, w2_ref[...])
    y2_ref[...] = dis_ref[...] * hw2


def _final_body(agg2p_ref, y2_ref, dis_ref, b2_ref, out_ref):
    agg = agg2p_ref[0] + agg2p_ref[1] + y2_ref[...]
    out_ref[...] = dis_ref[...] * agg + b2_ref[...]


def _tc_call(body, out_shapes):
    return pl.pallas_call(body, out_shape=out_shapes)


def kernel(x, edge_index, W1, b1, W2, b2):
    n, d = x.shape
    h = W1.shape[1]
    e = edge_index.shape[1]

    npad = ((n + NS * CHUNK - 1) // (NS * CHUNK)) * (NS * CHUNK)
    ch = ((e + NW * CHUNK * KGRP - 1) // (NW * CHUNK * KGRP)) * KGRP
    ep = NW * ch * CHUNK

    src = edge_index[0]
    dst = edge_index[1]
    if ep > e:
        pad = jnp.full((ep - e,), n, dtype=jnp.int32)
        src = jnp.concatenate([src, pad])
        dst = jnp.concatenate([dst, pad])
    src3 = src.reshape(NW, ch, CHUNK)
    dst3 = dst.reshape(NW, ch, CHUNK)

    xp = jnp.pad(x, ((0, npad - n), (0, 0)))
    zeros1 = jnp.zeros((npad,), _F32)
    zerosf = jnp.zeros((npad, h), _F32)
    ones = jnp.ones((CHUNK,), _F32)

    degp = _make_deg_kernel(ch, npad)(dst3, zeros1, ones)
    degp = degp.reshape(NC, npad, 1)

    y1, dis = _tc_call(_dense1_body, [
        jax.ShapeDtypeStruct((npad, h), _F32),
        jax.ShapeDtypeStruct((npad, 1), _F32),
    ])(xp, degp, W1)

    aggp = _make_agg_kernel(ch, npad, h)(src3, dst3, y1, zerosf)
    aggp = aggp.reshape(NC, npad, h)

    (y2,) = _tc_call(_dense2_body, [jax.ShapeDtypeStruct((npad, 1), _F32)])(
        aggp, y1, dis, b1.reshape(1, h), W2)

    agg2p = _make_agg_kernel(ch, npad, 1)(src3, dst3, y2.reshape(npad), zeros1)
    agg2p = agg2p.reshape(NC, npad, 1)

    (out,) = _tc_call(_final_body, [jax.ShapeDtypeStruct((npad, 1), _F32)])(
        agg2p, y2, dis, b2.reshape(1, 1))

    return out[:n, 0]


# speedup vs baseline: 41.6171x; 1.0202x over previous
import functools

import jax
import jax.numpy as jnp
from jax import lax
from jax.experimental import pallas as pl
from jax.experimental.pallas import tpu as pltpu
from jax.experimental.pallas import tpu_sc as plsc

NC = 2
NS = 16
NW = NC * NS
CHUNK = 128
NQ = 4

_F32 = jnp.float32


def _sc_mesh():
    return plsc.VectorSubcoreMesh(
        core_axis_name="c", subcore_axis_name="s", num_cores=NC, num_subcores=NS
    )


@functools.lru_cache(maxsize=None)
def _make_deg_kernel(qc: int, npad: int):
    rows = npad // NS

    def body(dst_hbm, zeros_hbm, ones_hbm, out_hbm,
             dstv, onesv, deg_sh, sem0, sem1):
        c = lax.axis_index("c")
        s = lax.axis_index("s")
        w = c * NS + s
        pltpu.sync_copy(zeros_hbm.at[pl.ds(s * rows, rows)],
                        deg_sh.at[pl.ds(s * rows, rows)])
        pltpu.sync_copy(ones_hbm, onesv)
        pltpu.sync_copy(dst_hbm.at[w], dstv)
        plsc.subcore_barrier()

        sems = (sem0, sem1)
        sd = [None] * NQ
        for q in range(NQ):
            if q >= 2:
                sd[q - 2].wait()
            sd[q] = pltpu.async_copy(onesv, deg_sh.at[dstv.at[q]],
                                     sems[q % 2], add=True)
        for q in range(max(0, NQ - 2), NQ):
            sd[q].wait()

        plsc.subcore_barrier()
        pltpu.sync_copy(deg_sh.at[pl.ds(s * rows, rows)], out_hbm.at[c, s])

    return pl.kernel(
        body,
        out_type=jax.ShapeDtypeStruct((NC, NS, rows), _F32),
        mesh=_sc_mesh(),
        scratch_types=[
            pltpu.VMEM((NQ, qc * CHUNK), jnp.int32),
            pltpu.VMEM((qc * CHUNK,), _F32),
            pltpu.VMEM_SHARED((npad,), _F32),
            pltpu.SemaphoreType.DMA,
            pltpu.SemaphoreType.DMA,
        ],
        compiler_params=pltpu.CompilerParams(use_tc_tiling_on_sc=False),
    )


@functools.lru_cache(maxsize=None)
def _make_agg_kernel(qc: int, npad: int, feat: int):
    rows = npad // NS
    vec = feat > 1

    def body(src_hbm, dst_hbm, y_hbm, zeros_hbm, out_hbm,
             srcv, dstv, buf0, buf1, agg_sh, gsem0, gsem1, ssem0, ssem1):
        c = lax.axis_index("c")
        s = lax.axis_index("s")
        w = c * NS + s
        pltpu.sync_copy(zeros_hbm.at[pl.ds(s * rows, rows)],
                        agg_sh.at[pl.ds(s * rows, rows)])
        pltpu.sync_copy(src_hbm.at[w], srcv)
        pltpu.sync_copy(dst_hbm.at[w], dstv)
        plsc.subcore_barrier()

        bufs = (buf0, buf1)
        gsems = (gsem0, gsem1)
        ssems = (ssem0, ssem1)

        def gstart(q):
            return pltpu.async_copy(y_hbm.at[srcv.at[q]], bufs[q % 2],
                                    gsems[q % 2])

        def sstart(q):
            return pltpu.async_copy(bufs[q % 2], agg_sh.at[dstv.at[q]],
                                    ssems[q % 2], add=True)

        gd = [None] * NQ
        sd = [None] * NQ
        gd[0] = gstart(0)
        if NQ > 1:
            gd[1] = gstart(1)
        for q in range(NQ):
            gd[q].wait()
            sd[q] = sstart(q)
            if q >= 1 and q + 1 < NQ:
                sd[q - 1].wait()
                gd[q + 1] = gstart(q + 1)
        for q in range(max(0, NQ - 2), NQ):
            sd[q].wait()

        plsc.subcore_barrier()
        pltpu.sync_copy(agg_sh.at[pl.ds(s * rows, rows)], out_hbm.at[c, s])

    out_shape = (NC, NS, rows, feat) if vec else (NC, NS, rows)
    buf_shape = (qc * CHUNK, feat) if vec else (qc * CHUNK,)
    return pl.kernel(
        body,
        out_type=jax.ShapeDtypeStruct(out_shape, _F32),
        mesh=_sc_mesh(),
        scratch_types=[
            pltpu.VMEM((NQ, qc * CHUNK), jnp.int32),
            pltpu.VMEM((NQ, qc * CHUNK), jnp.int32),
            pltpu.VMEM(buf_shape, _F32),
            pltpu.VMEM(buf_shape, _F32),
            pltpu.VMEM_SHARED((npad, feat) if vec else (npad,), _F32),
            pltpu.SemaphoreType.DMA,
            pltpu.SemaphoreType.DMA,
            pltpu.SemaphoreType.DMA,
            pltpu.SemaphoreType.DMA,
        ],
        compiler_params=pltpu.CompilerParams(use_tc_tiling_on_sc=False),
    )


def _dense1_body(x_ref, degp_ref, w1_ref, y_ref, dis_ref):
    deg = degp_ref[0] + degp_ref[1] + 1.0
    dis = lax.rsqrt(deg)
    xw = jnp.dot(x_ref[...], w1_ref[...])
    dis_ref[...] = dis
    y_ref[...] = dis * xw


def _dense2_body(aggp_ref, y1_ref, dis_ref, b1_ref, w2_ref, y2_ref):
    agg = aggp_ref[0] + aggp_ref[1] + y1_ref[...]
    out1 = dis_ref[...] * agg + b1_ref[...]
    h = jnp.maximum(out1, 0.0)
    hw2 = jnp.dot(h, w2_ref[...])
    y2_ref[...] = dis_ref[...] * hw2


def _final_body(agg2p_ref, y2_ref, dis_ref, b2_ref, out_ref):
    agg = agg2p_ref[0] + agg2p_ref[1] + y2_ref[...]
    out_ref[...] = dis_ref[...] * agg + b2_ref[...]


def _tc_call(body, out_shapes):
    return pl.pallas_call(body, out_shape=out_shapes)


def kernel(x, edge_index, W1, b1, W2, b2):
    n, d = x.shape
    h = W1.shape[1]
    e = edge_index.shape[1]

    npad = ((n + NS * CHUNK - 1) // (NS * CHUNK)) * (NS * CHUNK)
    unit = NW * CHUNK * NQ
    qc = ((e + unit - 1) // unit)
    ep = unit * qc

    src = edge_index[0]
    dst = edge_index[1]
    if ep > e:
        pad = jnp.full((ep - e,), n, dtype=jnp.int32)
        src = jnp.concatenate([src, pad])
        dst = jnp.concatenate([dst, pad])
    src4 = src.reshape(NW, NQ, qc * CHUNK)
    dst4 = dst.reshape(NW, NQ, qc * CHUNK)

    xp = jnp.pad(x, ((0, npad - n), (0, 0)))
    zeros1 = jnp.zeros((npad,), _F32)
    zerosf = jnp.zeros((npad, h), _F32)
    ones = jnp.ones((qc * CHUNK,), _F32)

    degp = _make_deg_kernel(qc, npad)(dst4, zeros1, ones)
    degp = degp.reshape(NC, npad, 1)

    y1, dis = _tc_call(_dense1_body, [
        jax.ShapeDtypeStruct((npad, h), _F32),
        jax.ShapeDtypeStruct((npad, 1), _F32),
    ])(xp, degp, W1)

    aggp = _make_agg_kernel(qc, npad, h)(src4, dst4, y1, zerosf)
    aggp = aggp.reshape(NC, npad, h)

    (y2,) = _tc_call(_dense2_body, [jax.ShapeDtypeStruct((npad, 1), _F32)])(
        aggp, y1, dis, b1.reshape(1, h), W2)

    agg2p = _make_agg_kernel(qc, npad, 1)(src4, dst4, y2.reshape(npad), zeros1)
    agg2p = agg2p.reshape(NC, npad, 1)

    (out,) = _tc_call(_final_body, [jax.ShapeDtypeStruct((npad, 1), _F32)])(
        agg2p, y2, dis, b2.reshape(1, 1))

    return out[:n, 0]


# speedup vs baseline: 48.0348x; 1.1542x over previous
import functools

import jax
import jax.numpy as jnp
from jax import lax
from jax.experimental import pallas as pl
from jax.experimental.pallas import tpu as pltpu
from jax.experimental.pallas import tpu_sc as plsc

NC = 2
NS = 16
NW = NC * NS
CHUNK = 128
NQ = 4

_F32 = jnp.float32


def _sc_mesh():
    return plsc.VectorSubcoreMesh(
        core_axis_name="c", subcore_axis_name="s", num_cores=NC, num_subcores=NS
    )


def _combine_tile_partials(s_dyn, accv, tmpv, resv, part_sh, npad):
    rows = npad // NS
    pltpu.sync_copy(accv, part_sh.at[s_dyn])
    plsc.subcore_barrier()
    pltpu.sync_copy(part_sh.at[:, pl.ds(s_dyn * rows, rows)], tmpv)
    for k in range(rows // 16):
        acc = tmpv[0, pl.ds(k * 16, 16)]
        for p in range(1, NS):
            acc = acc + tmpv[p, pl.ds(k * 16, 16)]
        resv[pl.ds(k * 16, 16)] = acc


@functools.lru_cache(maxsize=None)
def _make_deg_kernel(ept: int, npad: int):
    rows = npad // NS

    def body(dst_hbm, zeros_hbm, out_hbm,
             dstv, accv, tmpv, resv, part_sh):
        c = lax.axis_index("c")
        s = lax.axis_index("s")
        w = c * NS + s
        pltpu.sync_copy(zeros_hbm, accv)
        pltpu.sync_copy(dst_hbm.at[w], dstv)
        ones16 = jnp.ones((16,), _F32)

        @pl.loop(0, ept // 16, unroll=8)
        def _(i):
            dv = dstv[pl.ds(i * 16, 16)]
            plsc.addupdate_scatter(accv, [dv], ones16)

        _combine_tile_partials(s, accv, tmpv, resv, part_sh, npad)
        pltpu.sync_copy(resv, out_hbm.at[c, s])

    return pl.kernel(
        body,
        out_type=jax.ShapeDtypeStruct((NC, NS, rows), _F32),
        mesh=_sc_mesh(),
        scratch_types=[
            pltpu.VMEM((ept,), jnp.int32),
            pltpu.VMEM((npad,), _F32),
            pltpu.VMEM((NS, rows), _F32),
            pltpu.VMEM((rows,), _F32),
            pltpu.VMEM_SHARED((NS, npad), _F32),
        ],
        compiler_params=pltpu.CompilerParams(use_tc_tiling_on_sc=False,
                                             needs_layout_passes=False),
    )


@functools.lru_cache(maxsize=None)
def _make_agg1d_kernel(ept: int, npad: int):
    rows = npad // NS

    def body(src_hbm, dst_hbm, y_hbm, zeros_hbm, out_hbm,
             srcv, dstv, yv, accv, tmpv, resv, part_sh):
        c = lax.axis_index("c")
        s = lax.axis_index("s")
        w = c * NS + s
        pltpu.sync_copy(zeros_hbm, accv)
        pltpu.sync_copy(y_hbm, yv)
        pltpu.sync_copy(src_hbm.at[w], srcv)
        pltpu.sync_copy(dst_hbm.at[w], dstv)

        @pl.loop(0, ept // 16, unroll=8)
        def _(i):
            sv = srcv[pl.ds(i * 16, 16)]
            dv = dstv[pl.ds(i * 16, 16)]
            g = plsc.load_gather(yv, [sv])
            plsc.addupdate_scatter(accv, [dv], g)

        _combine_tile_partials(s, accv, tmpv, resv, part_sh, npad)
        pltpu.sync_copy(resv, out_hbm.at[c, s])

    return pl.kernel(
        body,
        out_type=jax.ShapeDtypeStruct((NC, NS, rows), _F32),
        mesh=_sc_mesh(),
        scratch_types=[
            pltpu.VMEM((ept,), jnp.int32),
            pltpu.VMEM((ept,), jnp.int32),
            pltpu.VMEM((npad,), _F32),
            pltpu.VMEM((npad,), _F32),
            pltpu.VMEM((NS, rows), _F32),
            pltpu.VMEM((rows,), _F32),
            pltpu.VMEM_SHARED((NS, npad), _F32),
        ],
        compiler_params=pltpu.CompilerParams(use_tc_tiling_on_sc=False,
                                             needs_layout_passes=False),
    )


@functools.lru_cache(maxsize=None)
def _make_agg_kernel(qc: int, npad: int, feat: int):
    rows = npad // NS
    vec = feat > 1

    def body(src_hbm, dst_hbm, y_hbm, zeros_hbm, out_hbm,
             srcv, dstv, buf0, buf1, agg_sh, gsem0, gsem1, ssem0, ssem1):
        c = lax.axis_index("c")
        s = lax.axis_index("s")
        w = c * NS + s
        pltpu.sync_copy(zeros_hbm.at[pl.ds(s * rows, rows)],
                        agg_sh.at[pl.ds(s * rows, rows)])
        pltpu.sync_copy(src_hbm.at[w], srcv)
        pltpu.sync_copy(dst_hbm.at[w], dstv)
        plsc.subcore_barrier()

        bufs = (buf0, buf1)
        gsems = (gsem0, gsem1)
        ssems = (ssem0, ssem1)

        def gstart(q):
            return pltpu.async_copy(y_hbm.at[srcv.at[q]], bufs[q % 2],
                                    gsems[q % 2])

        def sstart(q):
            return pltpu.async_copy(bufs[q % 2], agg_sh.at[dstv.at[q]],
                                    ssems[q % 2], add=True)

        gd = [None] * NQ
        sd = [None] * NQ
        gd[0] = gstart(0)
        if NQ > 1:
            gd[1] = gstart(1)
        for q in range(NQ):
            gd[q].wait()
            sd[q] = sstart(q)
            if q >= 1 and q + 1 < NQ:
                sd[q - 1].wait()
                gd[q + 1] = gstart(q + 1)
        for q in range(max(0, NQ - 2), NQ):
            sd[q].wait()

        plsc.subcore_barrier()
        pltpu.sync_copy(agg_sh.at[pl.ds(s * rows, rows)], out_hbm.at[c, s])

    out_shape = (NC, NS, rows, feat) if vec else (NC, NS, rows)
    buf_shape = (qc * CHUNK, feat) if vec else (qc * CHUNK,)
    return pl.kernel(
        body,
        out_type=jax.ShapeDtypeStruct(out_shape, _F32),
        mesh=_sc_mesh(),
        scratch_types=[
            pltpu.VMEM((NQ, qc * CHUNK), jnp.int32),
            pltpu.VMEM((NQ, qc * CHUNK), jnp.int32),
            pltpu.VMEM(buf_shape, _F32),
            pltpu.VMEM(buf_shape, _F32),
            pltpu.VMEM_SHARED((npad, feat) if vec else (npad,), _F32),
            pltpu.SemaphoreType.DMA,
            pltpu.SemaphoreType.DMA,
            pltpu.SemaphoreType.DMA,
            pltpu.SemaphoreType.DMA,
        ],
        compiler_params=pltpu.CompilerParams(use_tc_tiling_on_sc=False),
    )


def _dense1_body(x_ref, degp_ref, w1_ref, y_ref, dis_ref):
    deg = degp_ref[0] + degp_ref[1] + 1.0
    dis = lax.rsqrt(deg)
    xw = jnp.dot(x_ref[...], w1_ref[...])
    dis_ref[...] = dis
    y_ref[...] = dis * xw


def _dense2_body(aggp_ref, y1_ref, dis_ref, b1_ref, w2_ref, y2_ref):
    agg = aggp_ref[0] + aggp_ref[1] + y1_ref[...]
    out1 = dis_ref[...] * agg + b1_ref[...]
    h = jnp.maximum(out1, 0.0)
    hw2 = jnp.dot(h, w2_ref[...])
    y2_ref[...] = dis_ref[...] * hw2


def _final_body(agg2p_ref, y2_ref, dis_ref, b2_ref, out_ref):
    agg = agg2p_ref[0] + agg2p_ref[1] + y2_ref[...]
    out_ref[...] = dis_ref[...] * agg + b2_ref[...]


def _tc_call(body, out_shapes):
    return pl.pallas_call(body, out_shape=out_shapes)


def kernel(x, edge_index, W1, b1, W2, b2):
    n, d = x.shape
    h = W1.shape[1]
    e = edge_index.shape[1]

    npad = ((n + NS * CHUNK - 1) // (NS * CHUNK)) * (NS * CHUNK)
    unit = NW * CHUNK * NQ
    qc = ((e + unit - 1) // unit)
    ep = unit * qc

    src = edge_index[0]
    dst = edge_index[1]
    if ep > e:
        pad = jnp.full((ep - e,), n, dtype=jnp.int32)
        src = jnp.concatenate([src, pad])
        dst = jnp.concatenate([dst, pad])
    src4 = src.reshape(NW, NQ, qc * CHUNK)
    dst4 = dst.reshape(NW, NQ, qc * CHUNK)
    ept = NQ * qc * CHUNK
    src2 = src.reshape(NW, ept)
    dst2 = dst.reshape(NW, ept)

    xp = jnp.pad(x, ((0, npad - n), (0, 0)))
    zeros1 = jnp.zeros((npad,), _F32)
    zerosf = jnp.zeros((npad, h), _F32)

    degp = _make_deg_kernel(ept, npad)(dst2, zeros1)
    degp = degp.reshape(NC, npad, 1)

    y1, dis = _tc_call(_dense1_body, [
        jax.ShapeDtypeStruct((npad, h), _F32),
        jax.ShapeDtypeStruct((npad, 1), _F32),
    ])(xp, degp, W1)

    aggp = _make_agg_kernel(qc, npad, h)(src4, dst4, y1, zerosf)
    aggp = aggp.reshape(NC, npad, h)

    (y2,) = _tc_call(_dense2_body, [jax.ShapeDtypeStruct((npad, 1), _F32)])(
        aggp, y1, dis, b1.reshape(1, h), W2)

    agg2p = _make_agg1d_kernel(ept, npad)(src2, dst2, y2.reshape(npad), zeros1)
    agg2p = agg2p.reshape(NC, npad, 1)

    (out,) = _tc_call(_final_body, [jax.ShapeDtypeStruct((npad, 1), _F32)])(
        agg2p, y2, dis, b2.reshape(1, 1))

    return out[:n, 0]


# speedup vs baseline: 66.2549x; 1.3793x over previous
import functools

import jax
import jax.numpy as jnp
from jax import lax
from jax.experimental import pallas as pl
from jax.experimental.pallas import tpu as pltpu
from jax.experimental.pallas import tpu_sc as plsc

NC = 2
NS = 16
NW = NC * NS
CHUNK = 128
NQ = 4

_F32 = jnp.float32


def _sc_mesh():
    return plsc.VectorSubcoreMesh(
        core_axis_name="c", subcore_axis_name="s", num_cores=NC, num_subcores=NS
    )


def _combine_tile_partials(s_dyn, accv, tmpv, resv, part_sh, npad):
    rows = npad // NS
    pltpu.sync_copy(accv, part_sh.at[s_dyn])
    plsc.subcore_barrier()
    pltpu.sync_copy(part_sh.at[:, pl.ds(s_dyn * rows, rows)], tmpv)
    for k in range(rows // 16):
        acc = tmpv[0, pl.ds(k * 16, 16)]
        for p in range(1, NS):
            acc = acc + tmpv[p, pl.ds(k * 16, 16)]
        resv[pl.ds(k * 16, 16)] = acc


@functools.lru_cache(maxsize=None)
def _make_deg_kernel(ept: int, npad: int):
    rows = npad // NS

    def body(dst_hbm, out_hbm, dstv, accv, tmpv, resv, part_sh):
        c = lax.axis_index("c")
        s = lax.axis_index("s")
        w = c * NS + s
        pltpu.sync_copy(dst_hbm.at[w], dstv)
        z16 = jnp.zeros((16,), _F32)

        @pl.loop(0, npad // 16, unroll=8)
        def _(i):
            accv[pl.ds(i * 16, 16)] = z16

        ones16 = jnp.ones((16,), _F32)

        @pl.loop(0, ept // 16, unroll=5)
        def _(i):
            dv = dstv[pl.ds(i * 16, 16)]
            plsc.addupdate_scatter(accv, [dv], ones16)

        _combine_tile_partials(s, accv, tmpv, resv, part_sh, npad)
        pltpu.sync_copy(resv, out_hbm.at[c, s])

    return pl.kernel(
        body,
        out_type=jax.ShapeDtypeStruct((NC, NS, rows), _F32),
        mesh=_sc_mesh(),
        scratch_types=[
            pltpu.VMEM((ept,), jnp.int32),
            pltpu.VMEM((npad,), _F32),
            pltpu.VMEM((NS, rows), _F32),
            pltpu.VMEM((rows,), _F32),
            pltpu.VMEM_SHARED((NS, npad), _F32),
        ],
        compiler_params=pltpu.CompilerParams(use_tc_tiling_on_sc=False,
                                             needs_layout_passes=False),
    )


@functools.lru_cache(maxsize=None)
def _make_agg1d_kernel(ept: int, npad: int):
    rows = npad // NS

    def body(src_hbm, dst_hbm, y_hbm, out_hbm,
             srcv, dstv, yv, accv, tmpv, resv, part_sh):
        c = lax.axis_index("c")
        s = lax.axis_index("s")
        w = c * NS + s
        pltpu.sync_copy(y_hbm, yv)
        pltpu.sync_copy(src_hbm.at[w], srcv)
        pltpu.sync_copy(dst_hbm.at[w], dstv)
        z16 = jnp.zeros((16,), _F32)

        @pl.loop(0, npad // 16, unroll=8)
        def _(i):
            accv[pl.ds(i * 16, 16)] = z16

        @pl.loop(0, ept // 16, unroll=5)
        def _(i):
            sv = srcv[pl.ds(i * 16, 16)]
            dv = dstv[pl.ds(i * 16, 16)]
            g = plsc.load_gather(yv, [sv])
            plsc.addupdate_scatter(accv, [dv], g)

        _combine_tile_partials(s, accv, tmpv, resv, part_sh, npad)
        pltpu.sync_copy(resv, out_hbm.at[c, s])

    return pl.kernel(
        body,
        out_type=jax.ShapeDtypeStruct((NC, NS, rows), _F32),
        mesh=_sc_mesh(),
        scratch_types=[
            pltpu.VMEM((ept,), jnp.int32),
            pltpu.VMEM((ept,), jnp.int32),
            pltpu.VMEM((npad,), _F32),
            pltpu.VMEM((npad,), _F32),
            pltpu.VMEM((NS, rows), _F32),
            pltpu.VMEM((rows,), _F32),
            pltpu.VMEM_SHARED((NS, npad), _F32),
        ],
        compiler_params=pltpu.CompilerParams(use_tc_tiling_on_sc=False,
                                             needs_layout_passes=False),
    )


@functools.lru_cache(maxsize=None)
def _make_agg_kernel(qlen: int, npad: int, feat: int):
    rows = npad // NS
    vec = feat > 1

    def body(src_hbm, dst_hbm, y_hbm, zeros_hbm, out_hbm,
             srcv, dstv, buf0, buf1, y_sh, agg_sh,
             gsem0, gsem1, ssem0, ssem1):
        c = lax.axis_index("c")
        s = lax.axis_index("s")
        w = c * NS + s
        pltpu.sync_copy(zeros_hbm.at[pl.ds(s * rows, rows)],
                        agg_sh.at[pl.ds(s * rows, rows)])
        pltpu.sync_copy(y_hbm.at[pl.ds(s * rows, rows)],
                        y_sh.at[pl.ds(s * rows, rows)])
        pltpu.sync_copy(src_hbm.at[w], srcv)
        pltpu.sync_copy(dst_hbm.at[w], dstv)
        plsc.subcore_barrier()

        bufs = (buf0, buf1)
        gsems = (gsem0, gsem1)
        ssems = (ssem0, ssem1)

        def gstart(q):
            return pltpu.async_copy(y_sh.at[srcv.at[q]], bufs[q % 2],
                                    gsems[q % 2])

        def sstart(q):
            return pltpu.async_copy(bufs[q % 2], agg_sh.at[dstv.at[q]],
                                    ssems[q % 2], add=True)

        gd = [None] * NQ
        sd = [None] * NQ
        gd[0] = gstart(0)
        if NQ > 1:
            gd[1] = gstart(1)
        for q in range(NQ):
            gd[q].wait()
            sd[q] = sstart(q)
            if q >= 1 and q + 1 < NQ:
                sd[q - 1].wait()
                gd[q + 1] = gstart(q + 1)
        for q in range(max(0, NQ - 2), NQ):
            sd[q].wait()

        plsc.subcore_barrier()
        pltpu.sync_copy(agg_sh.at[pl.ds(s * rows, rows)], out_hbm.at[c, s])

    out_shape = (NC, NS, rows, feat) if vec else (NC, NS, rows)
    buf_shape = (qlen, feat) if vec else (qlen,)
    return pl.kernel(
        body,
        out_type=jax.ShapeDtypeStruct(out_shape, _F32),
        mesh=_sc_mesh(),
        scratch_types=[
            pltpu.VMEM((NQ, qlen), jnp.int32),
            pltpu.VMEM((NQ, qlen), jnp.int32),
            pltpu.VMEM(buf_shape, _F32),
            pltpu.VMEM(buf_shape, _F32),
            pltpu.VMEM_SHARED((npad, feat) if vec else (npad,), _F32),
            pltpu.VMEM_SHARED((npad, feat) if vec else (npad,), _F32),
            pltpu.SemaphoreType.DMA,
            pltpu.SemaphoreType.DMA,
            pltpu.SemaphoreType.DMA,
            pltpu.SemaphoreType.DMA,
        ],
        compiler_params=pltpu.CompilerParams(use_tc_tiling_on_sc=False),
    )


def _dense1_body(x_ref, degp_ref, w1_ref, y_ref, dis_ref):
    n = x_ref.shape[0]
    npad, h = y_ref.shape
    deg = degp_ref[0] + degp_ref[1] + 1.0
    dis = lax.rsqrt(deg)
    xw = jnp.dot(x_ref[...], w1_ref[...])
    dis_ref[...] = dis
    y_ref[0:n, :] = dis[0:n, :] * xw
    if npad > n:
        y_ref[n:npad, :] = jnp.zeros((npad - n, h), _F32)


def _dense2_body(aggp_ref, y1_ref, dis_ref, b1_ref, w2_ref, y2_ref):
    agg = aggp_ref[0] + aggp_ref[1] + y1_ref[...]
    out1 = dis_ref[...] * agg + b1_ref[...]
    h = jnp.maximum(out1, 0.0)
    hw2 = jnp.dot(h, w2_ref[...])
    y2_ref[...] = dis_ref[...] * hw2


def _final_body(agg2p_ref, y2_ref, dis_ref, b2_ref, out_ref):
    agg = agg2p_ref[0] + agg2p_ref[1] + y2_ref[...]
    out_ref[...] = dis_ref[...] * agg + b2_ref[...]


def _tc_call(body, out_shapes):
    return pl.pallas_call(body, out_shape=out_shapes)


def kernel(x, edge_index, W1, b1, W2, b2):
    n, d = x.shape
    h = W1.shape[1]
    e = edge_index.shape[1]

    npad = ((n + NS * CHUNK - 1) // (NS * CHUNK)) * (NS * CHUNK)
    unit = 16 * NQ
    ept = ((e + NW * unit - 1) // (NW * unit)) * unit
    ep = NW * ept
    qlen = ept // NQ

    src = edge_index[0]
    dst = edge_index[1]
    if ep > e:
        pad = jnp.full((ep - e,), n, dtype=jnp.int32)
        src = jnp.concatenate([src, pad])
        dst = jnp.concatenate([dst, pad])
    src4 = src.reshape(NW, NQ, qlen)
    dst4 = dst.reshape(NW, NQ, qlen)
    src2 = src.reshape(NW, ept)
    dst2 = dst.reshape(NW, ept)

    zerosf = jnp.zeros((npad, h), _F32)

    degp = _make_deg_kernel(ept, npad)(dst2)
    degp = degp.reshape(NC, npad, 1)

    y1, dis = _tc_call(_dense1_body, [
        jax.ShapeDtypeStruct((npad, h), _F32),
        jax.ShapeDtypeStruct((npad, 1), _F32),
    ])(x, degp, W1)

    aggp = _make_agg_kernel(qlen, npad, h)(src4, dst4, y1, zerosf)
    aggp = aggp.reshape(NC, npad, h)

    (y2,) = _tc_call(_dense2_body, [jax.ShapeDtypeStruct((npad, 1), _F32)])(
        aggp, y1, dis, b1.reshape(1, h), W2)

    agg2p = _make_agg1d_kernel(ept, npad)(src2, dst2, y2.reshape(npad))
    agg2p = agg2p.reshape(NC, npad, 1)

    (out,) = _tc_call(_final_body, [jax.ShapeDtypeStruct((npad, 1), _F32)])(
        agg2p, y2, dis, b2.reshape(1, 1))

    return out[:n, 0]


# speedup vs baseline: 79.4774x; 1.1996x over previous
import functools

import jax
import jax.numpy as jnp
from jax import lax
from jax.experimental import pallas as pl
from jax.experimental.pallas import tpu as pltpu
from jax.experimental.pallas import tpu_sc as plsc

NC = 2
NS = 16
NW = NC * NS
CHUNK = 128
NQ = 5

_F32 = jnp.float32


def _sc_mesh():
    return plsc.VectorSubcoreMesh(
        core_axis_name="c", subcore_axis_name="s", num_cores=NC, num_subcores=NS
    )


def _combine_tile_partials(s_dyn, accv, tmpv, resv, part_sh, npad):
    rows = npad // NS
    pltpu.sync_copy(accv, part_sh.at[s_dyn])
    plsc.subcore_barrier()
    pltpu.sync_copy(part_sh.at[:, pl.ds(s_dyn * rows, rows)], tmpv)
    for k in range(rows // 16):
        acc = tmpv[0, pl.ds(k * 16, 16)]
        for p in range(1, NS):
            acc = acc + tmpv[p, pl.ds(k * 16, 16)]
        resv[pl.ds(k * 16, 16)] = acc


@functools.lru_cache(maxsize=None)
def _make_deg_kernel(ept: int, npad: int):
    rows = npad // NS

    def body(dst_hbm, out_hbm, dstv, accv, tmpv, resv, part_sh):
        c = lax.axis_index("c")
        s = lax.axis_index("s")
        w = c * NS + s
        pltpu.sync_copy(dst_hbm.at[w], dstv)
        z16 = jnp.zeros((16,), _F32)

        @pl.loop(0, npad // 16, unroll=8)
        def _(i):
            accv[pl.ds(i * 16, 16)] = z16

        ones16 = jnp.ones((16,), _F32)

        @pl.loop(0, ept // 16, unroll=25)
        def _(i):
            dv = dstv[pl.ds(i * 16, 16)]
            plsc.addupdate_scatter(accv, [dv], ones16)

        _combine_tile_partials(s, accv, tmpv, resv, part_sh, npad)
        pltpu.sync_copy(resv, out_hbm.at[c, s])

    return pl.kernel(
        body,
        out_type=jax.ShapeDtypeStruct((NC, NS, rows), _F32),
        mesh=_sc_mesh(),
        scratch_types=[
            pltpu.VMEM((ept,), jnp.int32),
            pltpu.VMEM((npad,), _F32),
            pltpu.VMEM((NS, rows), _F32),
            pltpu.VMEM((rows,), _F32),
            pltpu.VMEM_SHARED((NS, npad), _F32),
        ],
        compiler_params=pltpu.CompilerParams(use_tc_tiling_on_sc=False,
                                             needs_layout_passes=False),
    )


@functools.lru_cache(maxsize=None)
def _make_agg1d_kernel(ept: int, npad: int):
    rows = npad // NS

    def body(src_hbm, dst_hbm, y_hbm, dis_hbm, out_hbm,
             srcv, dstv, yv, disv, accv, tmpv, resv, part_sh):
        c = lax.axis_index("c")
        s = lax.axis_index("s")
        w = c * NS + s
        pltpu.sync_copy(y_hbm, yv)
        pltpu.sync_copy(dis_hbm.at[pl.ds(s * rows, rows)], disv)
        pltpu.sync_copy(src_hbm.at[w], srcv)
        pltpu.sync_copy(dst_hbm.at[w], dstv)
        z16 = jnp.zeros((16,), _F32)

        @pl.loop(0, npad // 16, unroll=8)
        def _(i):
            accv[pl.ds(i * 16, 16)] = z16

        @pl.loop(0, ept // 16, unroll=25)
        def _(i):
            sv = srcv[pl.ds(i * 16, 16)]
            dv = dstv[pl.ds(i * 16, 16)]
            g = plsc.load_gather(yv, [sv])
            plsc.addupdate_scatter(accv, [dv], g)

        _combine_tile_partials(s, accv, tmpv, resv, part_sh, npad)
        for k in range(rows // 16):
            ds16 = pl.ds(k * 16, 16)
            yslice = yv[pl.ds(s * rows + k * 16, 16)]
            resv[ds16] = disv[ds16] * (resv[ds16] + 0.5 * yslice)
        pltpu.sync_copy(resv, out_hbm.at[c, s])

    return pl.kernel(
        body,
        out_type=jax.ShapeDtypeStruct((NC, NS, rows), _F32),
        mesh=_sc_mesh(),
        scratch_types=[
            pltpu.VMEM((ept,), jnp.int32),
            pltpu.VMEM((ept,), jnp.int32),
            pltpu.VMEM((npad,), _F32),
            pltpu.VMEM((rows,), _F32),
            pltpu.VMEM((npad,), _F32),
            pltpu.VMEM((NS, rows), _F32),
            pltpu.VMEM((rows,), _F32),
            pltpu.VMEM_SHARED((NS, npad), _F32),
        ],
        compiler_params=pltpu.CompilerParams(use_tc_tiling_on_sc=False,
                                             needs_layout_passes=False),
    )


@functools.lru_cache(maxsize=None)
def _make_agg_kernel(qlen: int, npad: int, feat: int):
    rows = npad // NS
    vec = feat > 1

    def body(src_hbm, dst_hbm, y_hbm, zeros_hbm, out_hbm,
             srcv, dstv, buf0, buf1, y_sh, agg_sh,
             gsem0, gsem1, ssem0, ssem1):
        c = lax.axis_index("c")
        s = lax.axis_index("s")
        w = c * NS + s
        pltpu.sync_copy(zeros_hbm.at[pl.ds(s * rows, rows)],
                        agg_sh.at[pl.ds(s * rows, rows)])
        pltpu.sync_copy(y_hbm.at[pl.ds(s * rows, rows)],
                        y_sh.at[pl.ds(s * rows, rows)])
        pltpu.sync_copy(src_hbm.at[w], srcv)
        pltpu.sync_copy(dst_hbm.at[w], dstv)
        plsc.subcore_barrier()

        bufs = (buf0, buf1)
        gsems = (gsem0, gsem1)
        ssems = (ssem0, ssem1)

        def gstart(q):
            return pltpu.async_copy(y_sh.at[srcv.at[q]], bufs[q % 2],
                                    gsems[q % 2])

        def sstart(q):
            return pltpu.async_copy(bufs[q % 2], agg_sh.at[dstv.at[q]],
                                    ssems[q % 2], add=True)

        gd = [None] * NQ
        sd = [None] * NQ
        gd[0] = gstart(0)
        if NQ > 1:
            gd[1] = gstart(1)
        for q in range(NQ):
            gd[q].wait()
            sd[q] = sstart(q)
            if q >= 1 and q + 1 < NQ:
                sd[q - 1].wait()
                gd[q + 1] = gstart(q + 1)
        for q in range(max(0, NQ - 2), NQ):
            sd[q].wait()

        plsc.subcore_barrier()
        pltpu.sync_copy(agg_sh.at[pl.ds(s * rows, rows)], out_hbm.at[c, s])

    out_shape = (NC, NS, rows, feat) if vec else (NC, NS, rows)
    buf_shape = (qlen, feat) if vec else (qlen,)
    return pl.kernel(
        body,
        out_type=jax.ShapeDtypeStruct(out_shape, _F32),
        mesh=_sc_mesh(),
        scratch_types=[
            pltpu.VMEM((NQ, qlen), jnp.int32),
            pltpu.VMEM((NQ, qlen), jnp.int32),
            pltpu.VMEM(buf_shape, _F32),
            pltpu.VMEM(buf_shape, _F32),
            pltpu.VMEM_SHARED((npad, feat) if vec else (npad,), _F32),
            pltpu.VMEM_SHARED((npad, feat) if vec else (npad,), _F32),
            pltpu.SemaphoreType.DMA,
            pltpu.SemaphoreType.DMA,
            pltpu.SemaphoreType.DMA,
            pltpu.SemaphoreType.DMA,
        ],
        compiler_params=pltpu.CompilerParams(use_tc_tiling_on_sc=False),
    )


def _dense1_body(x_ref, degp_ref, w1_ref, y_ref, dis_ref):
    n = x_ref.shape[0]
    npad, h = y_ref.shape
    deg = degp_ref[0] + degp_ref[1] + 1.0
    dis = lax.rsqrt(deg)
    xw = jnp.dot(x_ref[...], w1_ref[...])
    dis_ref[...] = dis
    y_ref[0:n, :] = dis[0:n, :] * xw
    if npad > n:
        y_ref[n:npad, :] = jnp.zeros((npad - n, h), _F32)


def _dense2_body(aggp_ref, y1_ref, dis_ref, b1_ref, w2_ref, y2_ref):
    agg = aggp_ref[0] + aggp_ref[1] + y1_ref[...]
    out1 = dis_ref[...] * agg + b1_ref[...]
    h = jnp.maximum(out1, 0.0)
    hw2 = jnp.dot(h, w2_ref[...])
    y2_ref[...] = dis_ref[...] * hw2


def _final_body(agg2p_ref, y2_ref, dis_ref, b2_ref, out_ref):
    agg = agg2p_ref[0] + agg2p_ref[1] + y2_ref[...]
    out_ref[...] = dis_ref[...] * agg + b2_ref[...]


def _tc_call(body, out_shapes):
    return pl.pallas_call(body, out_shape=out_shapes)


def kernel(x, edge_index, W1, b1, W2, b2):
    n, d = x.shape
    h = W1.shape[1]
    e = edge_index.shape[1]

    npad = ((n + NS * CHUNK - 1) // (NS * CHUNK)) * (NS * CHUNK)
    unit = 16 * NQ
    ept = ((e + NW * unit - 1) // (NW * unit)) * unit
    ep = NW * ept
    qlen = ept // NQ

    src = edge_index[0]
    dst = edge_index[1]
    if ep > e:
        pad = jnp.full((ep - e,), n, dtype=jnp.int32)
        src = jnp.concatenate([src, pad])
        dst = jnp.concatenate([dst, pad])
    src4 = src.reshape(NW, NQ, qlen)
    dst4 = dst.reshape(NW, NQ, qlen)
    src2 = src.reshape(NW, ept)
    dst2 = dst.reshape(NW, ept)

    zerosf = jnp.zeros((npad, h), _F32)

    degp = _make_deg_kernel(ept, npad)(dst2)
    degp = degp.reshape(NC, npad, 1)

    y1, dis = _tc_call(_dense1_body, [
        jax.ShapeDtypeStruct((npad, h), _F32),
        jax.ShapeDtypeStruct((npad, 1), _F32),
    ])(x, degp, W1)

    aggp = _make_agg_kernel(qlen, npad, h)(src4, dst4, y1, zerosf)
    aggp = aggp.reshape(NC, npad, h)

    (y2,) = _tc_call(_dense2_body, [jax.ShapeDtypeStruct((npad, 1), _F32)])(
        aggp, y1, dis, b1.reshape(1, h), W2)

    agg2p = _make_agg1d_kernel(ept, npad)(
        src2, dst2, y2.reshape(npad), dis.reshape(npad))
    outp = agg2p.reshape(NC, npad)

    return (outp[0] + outp[1] + b2[0])[:n]
